# baseline (speedup 1.0000x reference)
import jax
import jax.numpy as jnp
from jax import lax
from jax.experimental import pallas as pl

_R, _N = 128, 32768
_BR = 64


def _onehot_body(x_ref, o_ref):
    x = x_ref[...]
    m = jnp.max(x, axis=1, keepdims=True)
    iota = lax.broadcasted_iota(jnp.int32, x.shape, 1)
    first = jnp.min(jnp.where(x == m, iota, _N), axis=1, keepdims=True)
    o_ref[...] = iota == first


def kernel(probs):
    return pl.pallas_call(
        _onehot_body,
        grid=(_R // _BR,),
        in_specs=[pl.BlockSpec((_BR, _N), lambda i: (i, 0))],
        out_specs=pl.BlockSpec((_BR, _N), lambda i: (i, 0)),
        out_shape=jax.ShapeDtypeStruct((_R, _N), jnp.bool_),
    )(probs)


# speedup vs baseline: 1.2375x; 1.2375x over previous
import jax
import jax.numpy as jnp
from jax import lax
from jax.experimental import pallas as pl

_R, _N = 128, 32768
_BR = 64


def _onehot_body(x_ref, o_ref):
    x = x_ref[...]
    m = jnp.max(x, axis=1, keepdims=True)
    iota = lax.broadcasted_iota(jnp.int32, x.shape, 1)
    first = jnp.min(jnp.where(x == m, iota, _N), axis=1, keepdims=True)
    o_ref[...] = (iota == first).astype(jnp.int8)


def kernel(probs):
    oh8 = pl.pallas_call(
        _onehot_body,
        grid=(_R // _BR,),
        in_specs=[pl.BlockSpec((_BR, _N), lambda i: (i, 0))],
        out_specs=pl.BlockSpec((_BR, _N), lambda i: (i, 0)),
        out_shape=jax.ShapeDtypeStruct((_R, _N), jnp.int8),
    )(probs)
    return oh8.astype(jnp.bool_)
